# Initial kernel scaffold; baseline (speedup 1.0000x reference)
import jax
import jax.numpy as jnp
from jax.experimental import pallas as pl


def kernel(data, ids, space_pts, time_pts, query_pts, sW1, sb1, sW2, sb2, sF1, sFb1, sF2, sFb2, tW1, tb1, tW2, tb2, tF1, tFb1, tF2, tFb2, cW1, cb1, cW2, cb2, gW1, gb1, gW2, gb2, gF1, gFb1, gF2, gFb2):
    raise NotImplementedError("write your pallas kernel here")


# speedup vs baseline: 5.2458x; 5.2458x over previous
import functools

import jax
import jax.numpy as jnp
from jax import lax
from jax.experimental import pallas as pl

_BIG = 1e30


def _dotp(a, b):
    return jax.lax.dot(a, b, precision=jax.lax.Precision.HIGHEST,
                       preferred_element_type=jnp.float32)


def _min_and_onehot(d2, iota, n):
    minv = jnp.min(d2, axis=1, keepdims=True)
    idx = jnp.min(jnp.where(d2 == minv, iota, n), axis=1, keepdims=True)
    sel = iota == idx
    return sel, sel.astype(jnp.float32)


def _space_body(nq, n, k, cm, spT_ref, sp_ref, qp_ref, data_ref,
                w1_ref, b1_ref, w2_ref, b2_ref, f1p_ref, fb1_ref, f2_ref,
                fb2_ref, out_ref):
    spT = spT_ref[0]
    sp = sp_ref[0]
    qp = qp_ref[0]
    data = data_ref[0]
    pd = sp.shape[1]

    d2 = jnp.zeros((nq, n), jnp.float32)
    for d in range(pd):
        diff = qp[:, d:d + 1] - spT[d:d + 1, :]
        d2 = d2 + diff * diff
    iota = lax.broadcasted_iota(jnp.int32, (nq, n), 1)

    f = data.shape[1]
    a_parts = [jnp.zeros((nq, f), jnp.float32) for _ in range(cm)]
    for _ in range(k):
        sel, oh = _min_and_onehot(d2, iota, n)
        d2 = jnp.where(sel, _BIG, d2)
        nbp = jnp.dot(oh, sp, preferred_element_type=jnp.float32)
        rel = nbp - qp
        h = jax.nn.relu(_dotp(rel, w1_ref[...])
                        + b1_ref[...])
        w = _dotp(h, w2_ref[...]) + b2_ref[...]
        fk = jnp.dot(oh, data, preferred_element_type=jnp.float32)
        for m in range(cm):
            a_parts[m] = a_parts[m] + w[:, m:m + 1] * fk
    a = jnp.concatenate(a_parts, axis=1)
    h2 = jax.nn.relu(_dotp(a, f1p_ref[...]) + fb1_ref[...])
    out_ref[0] = _dotp(h2, f2_ref[...]) + fb2_ref[...]


def _time_enc(rel):
    parts = []
    for fn in (jnp.sin, jnp.cos):
        for i in range(4):
            parts.append(fn(rel * (float(2 ** i) * jnp.pi)))
    return jnp.concatenate(parts, axis=1)


def _time_body(nq, n, k, cm, ttT_ref, tp_ref, tq_ref, data_ref, snei_ref,
               w1_ref, b1_ref, w2_ref, b2_ref, f1pd_ref, f1pn_ref, fb1_ref,
               f2_ref, fb2_ref, cw1d_ref, cw1s_ref, cw1t_ref, cb1_ref,
               cw2_ref, cb2_ref, out_ref):
    ttT = ttT_ref[0]
    tp = tp_ref[0]
    tq = tq_ref[0]
    data = data_ref[0]
    snei = snei_ref[0]
    j = pl.program_id(1)

    diff = tq - ttT
    d2 = diff * diff
    iota = lax.broadcasted_iota(jnp.int32, (nq, n), 1)

    fd = data.shape[1]
    fn_ = snei.shape[1]
    ad = [jnp.zeros((nq, fd), jnp.float32) for _ in range(cm)]
    an = [jnp.zeros((nq, fn_), jnp.float32) for _ in range(cm)]
    for _ in range(k):
        sel, oh = _min_and_onehot(d2, iota, n)
        d2 = jnp.where(sel, _BIG, d2)
        nbp = jnp.dot(oh, tp, preferred_element_type=jnp.float32)
        enc = _time_enc(nbp - tq)
        h = jax.nn.relu(_dotp(enc, w1_ref[...])
                        + b1_ref[...])
        w = _dotp(h, w2_ref[...]) + b2_ref[...]
        fkd = jnp.dot(oh, data, preferred_element_type=jnp.float32)
        fkn = jnp.dot(oh, snei, preferred_element_type=jnp.float32)
        for m in range(cm):
            ad[m] = ad[m] + w[:, m:m + 1] * fkd
            an[m] = an[m] + w[:, m:m + 1] * fkn
    a_d = jnp.concatenate(ad, axis=1)
    a_n = jnp.concatenate(an, axis=1)
    h2 = jax.nn.relu(_dotp(a_d, f1pd_ref[...])
                     + _dotp(a_n, f1pn_ref[...])
                     + fb1_ref[...])
    tnei = _dotp(h2, f2_ref[...]) + fb2_ref[...]

    dblk = data_ref[0, pl.ds(j * nq, nq), :]
    sblk = snei_ref[0, pl.ds(j * nq, nq), :]
    hc = jax.nn.relu(
        _dotp(dblk, cw1d_ref[...])
        + _dotp(sblk, cw1s_ref[...])
        + _dotp(tnei, cw1t_ref[...])
        + cb1_ref[...])
    out_ref[0] = _dotp(hc, cw2_ref[...]) + cb2_ref[...]


def _query_body(nq, n, k, cm, ttT_ref, tp_ref, qq_ref, feats_ref,
                w1_ref, b1_ref, w2_ref, b2_ref, f1p_ref, fb1_ref, f2_ref,
                fb2_ref, out_ref):
    ttT = ttT_ref[0]
    tp = tp_ref[0]
    qq = qq_ref[0]
    feats = feats_ref[0]

    diff = qq - ttT
    d2 = diff * diff
    iota = lax.broadcasted_iota(jnp.int32, (nq, n), 1)

    f = feats.shape[1]
    a_parts = [jnp.zeros((nq, f), jnp.float32) for _ in range(cm)]
    for _ in range(k):
        sel, oh = _min_and_onehot(d2, iota, n)
        d2 = jnp.where(sel, _BIG, d2)
        nbp = jnp.dot(oh, tp, preferred_element_type=jnp.float32)
        enc = _time_enc(nbp - qq)
        h = jax.nn.relu(_dotp(enc, w1_ref[...])
                        + b1_ref[...])
        w = _dotp(h, w2_ref[...]) + b2_ref[...]
        fk = jnp.dot(oh, feats, preferred_element_type=jnp.float32)
        for m in range(cm):
            a_parts[m] = a_parts[m] + w[:, m:m + 1] * fk
    a = jnp.concatenate(a_parts, axis=1)
    h2 = jax.nn.relu(_dotp(a, f1p_ref[...]) + fb1_ref[...])
    out_ref[0] = _dotp(h2, f2_ref[...]) + fb2_ref[...]


def _perm_f1(f1, c, cm, scale):
    fh = f1.shape[1]
    return (f1.reshape(c, cm, fh).transpose(1, 0, 2).reshape(c * cm, fh)
            * scale)


def _full(shape):
    nd = len(shape)
    return pl.BlockSpec(shape, lambda b, j, _nd=nd: (0,) * _nd)


def _batch_full(shape):
    return pl.BlockSpec((1,) + shape[1:], lambda b, j: (b, 0, 0))


def _batch_block(shape, bq):
    return pl.BlockSpec((1, bq, shape[2]), lambda b, j: (b, j, 0))


def kernel(data, ids, space_pts, time_pts, query_pts, sW1, sb1, sW2, sb2,
           sF1, sFb1, sF2, sFb2, tW1, tb1, tW2, tb2, tF1, tFb1, tF2, tFb2,
           cW1, cb1, cW2, cb2, gW1, gb1, gW2, gb2, gF1, gFb1, gF2, gFb2):
    B, N, F = data.shape
    Q = query_pts.shape[1]
    PD = space_pts.shape[2]
    WH = sW1.shape[1]
    CM = sW2.shape[1]
    NSZ = sF2.shape[1]
    CM2 = gW2.shape[1]
    LS = cW2.shape[1]
    BQ = min(256, N)
    BQQ = min(256, Q)
    nblk = N // BQ
    qblk = Q // BQQ

    spT = space_pts.transpose(0, 2, 1)
    ttT = time_pts.transpose(0, 2, 1)
    sF1p = _perm_f1(sF1, F, CM, 1.0 / 16.0)
    tF1r = tF1.reshape(F + NSZ, CM, tF1.shape[1])
    tF1pd = _perm_f1(tF1r[:F].reshape(F * CM, -1), F, CM, 1.0 / 8.0)
    tF1pn = _perm_f1(tF1r[F:].reshape(NSZ * CM, -1), NSZ, CM, 1.0 / 8.0)
    gF1p = _perm_f1(gF1, LS, CM2, 1.0 / 8.0)
    r2 = lambda v: v.reshape(1, -1)

    space_nei = pl.pallas_call(
        functools.partial(_space_body, BQ, N, 16, CM),
        grid=(B, nblk),
        in_specs=[
            _batch_full(spT.shape),
            _batch_full(space_pts.shape),
            _batch_block(space_pts.shape, BQ),
            _batch_full(data.shape),
            _full(sW1.shape), _full((1, WH)), _full(sW2.shape),
            _full((1, CM)), _full(sF1p.shape), _full((1, sFb1.shape[0])),
            _full(sF2.shape), _full((1, NSZ)),
        ],
        out_specs=_batch_block((B, N, NSZ), BQ),
        out_shape=jax.ShapeDtypeStruct((B, N, NSZ), jnp.float32),
    )(spT, space_pts, space_pts, data, sW1, r2(sb1), sW2, r2(sb2), sF1p,
      r2(sFb1), sF2, r2(sFb2))

    space_in = pl.pallas_call(
        functools.partial(_time_body, BQ, N, 8, CM),
        grid=(B, nblk),
        in_specs=[
            _batch_full(ttT.shape),
            _batch_full(time_pts.shape),
            _batch_block(time_pts.shape, BQ),
            _batch_full(data.shape),
            _batch_full(space_nei.shape),
            _full(tW1.shape), _full((1, WH)), _full(tW2.shape),
            _full((1, CM)), _full(tF1pd.shape), _full(tF1pn.shape),
            _full((1, tFb1.shape[0])), _full(tF2.shape), _full((1, NSZ)),
            _full((F, cW1.shape[1])), _full((NSZ, cW1.shape[1])),
            _full((NSZ, cW1.shape[1])), _full((1, cb1.shape[0])),
            _full(cW2.shape), _full((1, LS)),
        ],
        out_specs=_batch_block((B, N, LS), BQ),
        out_shape=jax.ShapeDtypeStruct((B, N, LS), jnp.float32),
    )(ttT, time_pts, time_pts, data, space_nei, tW1, r2(tb1), tW2, r2(tb2),
      tF1pd, tF1pn, r2(tFb1), tF2, r2(tFb2), cW1[:F], cW1[F:F + NSZ],
      cW1[F + NSZ:], r2(cb1), cW2, r2(cb2))

    out = pl.pallas_call(
        functools.partial(_query_body, BQQ, N, 8, CM2),
        grid=(B, qblk),
        in_specs=[
            _batch_full(ttT.shape),
            _batch_full(time_pts.shape),
            _batch_block(query_pts.shape, BQQ),
            _batch_full(space_in.shape),
            _full(gW1.shape), _full((1, gW1.shape[1])), _full(gW2.shape),
            _full((1, CM2)), _full(gF1p.shape), _full((1, gFb1.shape[0])),
            _full(gF2.shape), _full((1, gF2.shape[1])),
        ],
        out_specs=_batch_block((B, Q, gF2.shape[1]), BQQ),
        out_shape=jax.ShapeDtypeStruct((B, Q, gF2.shape[1]), jnp.float32),
    )(ttT, time_pts, query_pts, space_in, gW1, r2(gb1), gW2, r2(gb2), gF1p,
      r2(gFb1), gF2, r2(gFb2))
    return out
